# baseline (speedup 1.0000x reference)
import jax
import jax.numpy as jnp
from jax.experimental import pallas as pl
from jax.experimental.pallas import tpu as pltpu

R = 1024
NH = 16
DH = 64
SCALE = 0.125


def _proj_body(xq_ref, xkv_ref, wq_ref, wk_ref, wv_ref, q_ref, k_ref, v_ref):
    xq = xq_ref[0].astype(jnp.bfloat16)
    xkv = xkv_ref[0].astype(jnp.bfloat16)
    q_ref[0] = jnp.dot(
        xq, wq_ref[...],
        preferred_element_type=jnp.float32).astype(jnp.bfloat16)
    k_ref[0] = jnp.dot(
        xkv, wk_ref[...],
        preferred_element_type=jnp.float32).astype(jnp.bfloat16)
    v_ref[0] = jnp.dot(
        xkv, wv_ref[...],
        preferred_element_type=jnp.float32).astype(jnp.bfloat16)


def _attn_body(q_ref, k_ref, v_ref, acc_ref):
    c = pl.program_id(1)
    coef = jnp.where(c == 0, 2.0 / 3.0, jnp.where(c == 1, 1.0 / 3.0, 1.0))
    accumulate = c == 1
    for h in range(NH):
        sl = slice(h * DH, (h + 1) * DH)
        q = q_ref[0, :, sl]
        k = k_ref[0, :, sl]
        v = v_ref[0, :, sl]
        s = jax.lax.dot_general(
            q, k, (((1,), (1,)), ((), ())),
            preferred_element_type=jnp.float32) * SCALE
        m = jnp.max(s, axis=1, keepdims=True)
        p = jnp.exp(s - m)
        l = jnp.sum(p, axis=1, keepdims=True)
        o = jnp.dot(p.astype(jnp.bfloat16), v,
                    preferred_element_type=jnp.float32)
        o = o * (coef / l)

        @pl.when(accumulate)
        def _():
            acc_ref[0, :, sl] += o

        @pl.when(jnp.logical_not(accumulate))
        def _():
            acc_ref[0, :, sl] = o


def _out_body(acc_ref, wo_ref, out_ref):
    out_ref[0] = jnp.dot(acc_ref[0].astype(jnp.bfloat16), wo_ref[...],
                         preferred_element_type=jnp.float32)


def kernel(x, Wq, Wk, Wv, Wo):
    B, N, D = x.shape
    f32 = jnp.float32
    bf16 = jnp.bfloat16
    Wq = Wq.astype(bf16)
    Wk = Wk.astype(bf16)
    Wv = Wv.astype(bf16)
    Wo = Wo.astype(bf16)
    RP = 512
    q, k, v = pl.pallas_call(
        _proj_body,
        grid=(B, (2 * R) // RP),
        in_specs=[
            pl.BlockSpec((1, RP, D), lambda b, j: (b, j, 0)),
            pl.BlockSpec((1, RP, D), lambda b, j: (b, j + R // RP, 0)),
            pl.BlockSpec((D, D), lambda b, j: (0, 0)),
            pl.BlockSpec((D, D), lambda b, j: (0, 0)),
            pl.BlockSpec((D, D), lambda b, j: (0, 0)),
        ],
        out_specs=[
            pl.BlockSpec((1, RP, D), lambda b, j: (b, j, 0)),
            pl.BlockSpec((1, RP, D), lambda b, j: (b, j, 0)),
            pl.BlockSpec((1, RP, D), lambda b, j: (b, j, 0)),
        ],
        out_shape=[jax.ShapeDtypeStruct((B, 2 * R, D), bf16)] * 3,
        compiler_params=pltpu.CompilerParams(
            dimension_semantics=("parallel", "arbitrary")),
    )(x, x, Wq, Wk, Wv)

    acc = pl.pallas_call(
        _attn_body,
        grid=(B, 3),
        in_specs=[
            pl.BlockSpec((1, R, D), lambda b, c: (b, c // 2, 0)),
            pl.BlockSpec((1, R, D), lambda b, c: (b, (c + 1) // 2, 0)),
            pl.BlockSpec((1, R, D), lambda b, c: (b, (c + 1) // 2, 0)),
        ],
        out_specs=pl.BlockSpec((1, R, D), lambda b, c: (b, c // 2, 0)),
        out_shape=jax.ShapeDtypeStruct((B, 2 * R, D), f32),
        compiler_params=pltpu.CompilerParams(
            dimension_semantics=("arbitrary", "arbitrary")),
    )(q, k, v)

    out01 = pl.pallas_call(
        _out_body,
        grid=(B, 2),
        in_specs=[
            pl.BlockSpec((1, R, D), lambda b, j: (b, j, 0)),
            pl.BlockSpec((D, D), lambda b, j: (0, 0)),
        ],
        out_specs=pl.BlockSpec((1, R, D), lambda b, j: (b, j, 0)),
        out_shape=jax.ShapeDtypeStruct((B, 2 * R, D), f32),
        compiler_params=pltpu.CompilerParams(
            dimension_semantics=("parallel", "arbitrary")),
    )(acc, Wo)

    return jnp.concatenate([out01, x[:, 2 * R:, :]], axis=1)


# speedup vs baseline: 1.0793x; 1.0793x over previous
import jax
import jax.numpy as jnp
from jax.experimental import pallas as pl
from jax.experimental.pallas import tpu as pltpu

R = 1024
NH = 16
DH = 64
SCALE = 0.125


def _fused_body(xs_ref, xd_ref, wq_ref, wk_ref, wv_ref, wo_ref, out_ref,
                q_s, k_s, v_s, acc_s):
    c = pl.program_id(1)
    bf16 = jnp.bfloat16
    f32 = jnp.float32

    @pl.when(c != 1)
    def _():
        q_s[...] = (jnp.dot(xs_ref[0].astype(bf16), wq_ref[...],
                            preferred_element_type=f32) * SCALE).astype(bf16)

    @pl.when(c != 2)
    def _():
        xd = xd_ref[0].astype(bf16)
        k_s[...] = jnp.dot(xd, wk_ref[...],
                           preferred_element_type=f32).astype(bf16)
        v_s[...] = jnp.dot(xd, wv_ref[...],
                           preferred_element_type=f32).astype(bf16)

    coef = jnp.where(c == 0, 2.0 / 3.0, jnp.where(c == 1, 1.0 / 3.0, 1.0))
    for h in range(NH):
        sl = slice(h * DH, (h + 1) * DH)
        q = q_s[:, sl]
        k = k_s[:, sl]
        v = v_s[:, sl]
        s = jax.lax.dot_general(q, k, (((1,), (1,)), ((), ())),
                                preferred_element_type=f32)
        m = jnp.max(s, axis=1, keepdims=True)
        p = jnp.exp(s - m)
        l = jnp.sum(p, axis=1, keepdims=True)
        o = jnp.dot(p.astype(bf16), v, preferred_element_type=f32)
        o = o * (coef / l)

        @pl.when(c == 1)
        def _():
            acc_s[:, sl] += o

        @pl.when(c != 1)
        def _():
            acc_s[:, sl] = o

    @pl.when(c >= 1)
    def _():
        out_ref[0] = jnp.dot(acc_s[...].astype(bf16), wo_ref[...],
                             preferred_element_type=f32)


def kernel(x, Wq, Wk, Wv, Wo):
    B, N, D = x.shape
    bf16 = jnp.bfloat16
    Wq = Wq.astype(bf16)
    Wk = Wk.astype(bf16)
    Wv = Wv.astype(bf16)
    Wo = Wo.astype(bf16)
    return pl.pallas_call(
        _fused_body,
        grid=(B, 3),
        in_specs=[
            pl.BlockSpec((1, R, D), lambda b, c: (b, c // 2, 0)),
            pl.BlockSpec((1, R, D), lambda b, c: (b, (c + 3) // 2, 0)),
            pl.BlockSpec((D, D), lambda b, c: (0, 0)),
            pl.BlockSpec((D, D), lambda b, c: (0, 0)),
            pl.BlockSpec((D, D), lambda b, c: (0, 0)),
            pl.BlockSpec((D, D), lambda b, c: (0, 0)),
        ],
        out_specs=pl.BlockSpec((1, R, D), lambda b, c: (b, c // 2, 0)),
        out_shape=jax.ShapeDtypeStruct((B, N, D), jnp.float32),
        scratch_shapes=[
            pltpu.VMEM((R, D), bf16),
            pltpu.VMEM((R, D), bf16),
            pltpu.VMEM((R, D), bf16),
            pltpu.VMEM((R, D), jnp.float32),
        ],
        input_output_aliases={0: 0},
        compiler_params=pltpu.CompilerParams(
            dimension_semantics=("arbitrary", "arbitrary")),
    )(x, x, Wq, Wk, Wv, Wo)
